# baseline (speedup 1.0000x reference)
import functools

import jax
import jax.numpy as jnp
import numpy as np
from jax import lax
from jax.experimental import pallas as pl
from jax.experimental.pallas import tpu as pltpu
from jax.experimental.pallas import tpu_sc as plsc

_ORDER_CACHE = {}


def _perm_order(n):
    if n not in _ORDER_CACHE:
        try:
            with jax.ensure_compile_time_eval():
                order = jax.random.permutation(jax.random.key(42), n)
            _ORDER_CACHE[n] = np.asarray(order, dtype=np.int32)
        except Exception:
            _ORDER_CACHE[n] = None
    const = _ORDER_CACHE[n]
    if const is not None:
        return jnp.asarray(const)
    return jax.random.permutation(jax.random.key(42), n).astype(jnp.int32)


@functools.lru_cache(maxsize=None)
def _build_gather(n, d, chunk=8, nbuf=4):
    rows = n
    info = plsc.get_sparse_core_info()
    nc, ns = info.num_cores, info.num_subcores
    nw = nc * ns
    assert rows % nw == 0
    b_per_w = rows // nw
    assert b_per_w % chunk == 0
    nchunks = b_per_w // chunk
    assert nchunks % nbuf == 0
    mesh = plsc.VectorSubcoreMesh(core_axis_name="c", subcore_axis_name="s")

    @functools.partial(
        pl.kernel,
        mesh=mesh,
        out_type=jax.ShapeDtypeStruct((n, d), jnp.float32),
        scratch_types=[
            pltpu.VMEM((b_per_w,), jnp.int32),
            pltpu.VMEM((nbuf * chunk, d), jnp.float32),
        ] + [pltpu.SemaphoreType.DMA] * (2 * nbuf),
    )
    def gather_kernel(x_hbm, idx_hbm, out_hbm, idx_v, buf, *sems):
        wid = lax.axis_index("s") * nc + lax.axis_index("c")
        base = wid * b_per_w
        pltpu.sync_copy(idx_hbm.at[pl.ds(base, b_per_w)], idx_v)
        gsem = sems[:nbuf]
        osem = sems[nbuf:]

        def bslice(b):
            return buf.at[pl.ds(b * chunk, chunk)]

        def gather_copy(c, b):
            return pltpu.make_async_copy(
                x_hbm.at[idx_v.at[pl.ds(c * chunk, chunk)]],
                bslice(b), gsem[b])

        def out_copy(c, b):
            return pltpu.make_async_copy(
                bslice(b), out_hbm.at[pl.ds(base + c * chunk, chunk)],
                osem[b])

        for k in range(nbuf - 1):
            gather_copy(k, k).start()

        @pl.loop(0, nchunks, step=nbuf)
        def _group(c0):
            for j in range(nbuf):
                s = c0 + j
                b = j
                pb = (j - 1) % nbuf
                gather_copy(s, b).wait()
                out_copy(s, b).start()

                @pl.when(s >= 1)
                def _free_prev():
                    out_copy(s - 1, pb).wait()

                @pl.when(s + nbuf - 1 < nchunks)
                def _refill():
                    gather_copy(s + nbuf - 1, pb).start()

        out_copy(nchunks - 1, (nchunks - 1) % nbuf).wait()

    return gather_kernel


def kernel(x):
    n = x.shape[0]
    d = 1
    for s in x.shape[1:]:
        d *= s
    order = _perm_order(n)
    out = _build_gather(n, d)(x.reshape(n, d), order)
    return out.reshape(x.shape)
